# Initial kernel scaffold; baseline (speedup 1.0000x reference)
import jax
import jax.numpy as jnp
from jax.experimental import pallas as pl


def kernel(x, edge_index, batch, W0, b0, W1, b1, W2, b2, Wl, bl):
    raise NotImplementedError("write your pallas kernel here")


# speedup vs baseline: 10.3220x; 10.3220x over previous
import functools

import jax
import jax.numpy as jnp
from jax import lax
from jax.experimental import pallas as pl
from jax.experimental.pallas import tpu as pltpu
from jax.experimental.pallas import tpu_sc as plsc

NC = 2
NS = 16
LANES = 16
CHUNK = 128


def _mesh():
    return plsc.VectorSubcoreMesh(core_axis_name="c", subcore_axis_name="s",
                                  num_cores=NC, num_subcores=NS)


def _sc_degree(dstm, n_pad):
    ec = dstm.shape[0]
    ec_half = ec // NC
    per = ec_half // NS
    rem = ec_half - per * NS
    slc = n_pad // NS

    @functools.partial(
        pl.kernel,
        out_type=jax.ShapeDtypeStruct((NC * n_pad,), jnp.float32),
        mesh=_mesh(),
        scratch_types=[
            pltpu.VMEM((n_pad,), jnp.float32),
            pltpu.VMEM((CHUNK,), jnp.int32),
            pltpu.VMEM((slc,), jnp.float32),
            pltpu.VMEM_SHARED((NS * n_pad,), jnp.float32),
        ],
        compiler_params=pltpu.CompilerParams(needs_layout_passes=False,
                                             use_tc_tiling_on_sc=False),
    )
    def k(dstm_hbm, out_hbm, hist, drow, rbuf, shared):
        c = lax.axis_index("c")
        t = lax.axis_index("s")

        zeros16 = jnp.zeros((LANES,), jnp.float32)
        ones16 = jnp.ones((LANES,), jnp.float32)

        def zbody(i, carry):
            hist[pl.ds(i * LANES, LANES)] = zeros16
            return carry
        lax.fori_loop(0, n_pad // LANES, zbody, 0)

        start = c * ec_half + per * t + jnp.minimum(t, rem)
        cnt = jnp.where(t < rem, per + 1, per)

        def cbody(ci, carry):
            pltpu.sync_copy(dstm_hbm.at[ci], drow)
            for g in range(CHUNK // LANES):
                idx = drow[pl.ds(g * LANES, LANES)]
                plsc.addupdate_scatter(hist, [idx], ones16)
            return carry
        lax.fori_loop(start, start + cnt, cbody, 0)

        pltpu.sync_copy(hist, shared.at[pl.ds(t * n_pad, n_pad)])
        plsc.subcore_barrier()

        base = t * slc
        pltpu.sync_copy(shared.at[pl.ds(base, slc)], hist.at[pl.ds(0, slc)])
        for j in range(1, NS):
            pltpu.sync_copy(shared.at[pl.ds(j * n_pad + base, slc)], rbuf)

            def abody(i, carry):
                o = i * LANES
                hist[pl.ds(o, LANES)] = hist[pl.ds(o, LANES)] + rbuf[pl.ds(o, LANES)]
                return carry
            lax.fori_loop(0, slc // LANES, abody, 0)
        pltpu.sync_copy(hist.at[pl.ds(0, slc)],
                        out_hbm.at[pl.ds(c * n_pad + base, slc)])

    return k(dstm).reshape(NC, n_pad)


def _sc_scatter(y_a, y_b, srcm, dstm, n_pad):
    ec = srcm.shape[0]
    per = ec // NS
    rem = ec - per * NS
    slc = n_pad // NS
    nfull = slc // CHUNK
    tail = slc - nfull * CHUNK

    @functools.partial(
        pl.kernel,
        out_type=(jax.ShapeDtypeStruct((n_pad, 32), jnp.float32),
                  jax.ShapeDtypeStruct((n_pad, 32), jnp.float32)),
        mesh=_mesh(),
        scratch_types=[
            pltpu.VMEM((CHUNK,), jnp.int32),
            pltpu.VMEM((CHUNK,), jnp.int32),
            pltpu.VMEM((CHUNK, 32), jnp.float32),
            pltpu.VMEM((CHUNK, 32), jnp.float32),
            pltpu.VMEM_SHARED((n_pad, 32), jnp.float32),
            pltpu.SemaphoreType.DMA,
        ],
        compiler_params=pltpu.CompilerParams(needs_layout_passes=False,
                                             use_tc_tiling_on_sc=False),
    )
    def k(ya_hbm, yb_hbm, srcm_hbm, dstm_hbm, out_a, out_b,
          srow, drow, rows, zbuf, acc, sem):
        c = lax.axis_index("c")
        t = lax.axis_index("s")

        zeros16 = jnp.zeros((LANES,), jnp.float32)

        def zbody(i, carry):
            r = i // 2
            zbuf[r, pl.ds((i % 2) * LANES, LANES)] = zeros16
            return carry
        lax.fori_loop(0, CHUNK * 2, zbody, 0)

        base = t * slc
        for q in range(nfull):
            pltpu.sync_copy(zbuf, acc.at[pl.ds(base + q * CHUNK, CHUNK)])
        if tail:
            pltpu.sync_copy(zbuf.at[pl.ds(0, tail)],
                            acc.at[pl.ds(base + nfull * CHUNK, tail)])
        plsc.subcore_barrier()

        start = per * t + jnp.minimum(t, rem)
        cnt = jnp.where(t < rem, per + 1, per)

        def body(ci, carry):
            pltpu.sync_copy(srcm_hbm.at[ci], srow)
            pltpu.sync_copy(dstm_hbm.at[ci], drow)

            @pl.when(c == 0)
            def _():
                pltpu.async_copy(ya_hbm.at[srow], rows, sem).wait()

            @pl.when(c == 1)
            def _():
                pltpu.async_copy(yb_hbm.at[srow], rows, sem).wait()

            pltpu.sync_copy(rows, acc.at[drow], add=True)
            return carry
        lax.fori_loop(start, start + cnt, body, 0)
        plsc.subcore_barrier()

        @pl.when(c == 0)
        def _():
            pltpu.sync_copy(acc.at[pl.ds(base, slc)], out_a.at[pl.ds(base, slc)])

        @pl.when(c == 1)
        def _():
            pltpu.sync_copy(acc.at[pl.ds(base, slc)], out_b.at[pl.ds(base, slc)])

    return k(y_a, y_b, srcm, dstm)


def _pick_nb(n):
    for nb in (2000, 2500, 1000, 500, 250, 200, 125, 100, 50, 25, 16, 8):
        if n % nb == 0:
            return nb
    return n


def _tc_first(x, degp, w0):
    n, din = x.shape
    h = w0.shape[1]
    nb = _pick_nb(n)

    def body(x_ref, dp_ref, w_ref, ya_ref, yb_ref, dinv_ref):
        deg = dp_ref[0] + dp_ref[1] + 1.0
        dinv = lax.rsqrt(deg)
        y = jnp.dot(x_ref[...], w_ref[...],
                    preferred_element_type=jnp.float32) * dinv
        ya_ref[...] = y[:, :32]
        yb_ref[...] = y[:, 32:]
        dinv_ref[...] = dinv

    return pl.pallas_call(
        body,
        grid=(n // nb,),
        in_specs=[
            pl.BlockSpec((nb, din), lambda i: (i, 0)),
            pl.BlockSpec((2, nb, 1), lambda i: (0, i, 0)),
            pl.BlockSpec((din, h), lambda i: (0, 0)),
        ],
        out_specs=[
            pl.BlockSpec((nb, 32), lambda i: (i, 0)),
            pl.BlockSpec((nb, 32), lambda i: (i, 0)),
            pl.BlockSpec((nb, 1), lambda i: (i, 0)),
        ],
        out_shape=[
            jax.ShapeDtypeStruct((n, 32), jnp.float32),
            jax.ShapeDtypeStruct((n, 32), jnp.float32),
            jax.ShapeDtypeStruct((n, 1), jnp.float32),
        ],
    )(x, degp, w0)


def _tc_mid(agg_a, agg_b, y_a, y_b, dinv, w, b_a, b_b):
    n = agg_a.shape[0]
    h = w.shape[0]
    nb = _pick_nb(n)

    def body(aa, ab, ya, yb, dv, w_ref, ba, bb, oa, ob):
        dinv = dv[...]
        ha = jnp.maximum((aa[...] + ya[...]) * dinv + ba[...], 0.0)
        hb = jnp.maximum((ab[...] + yb[...]) * dinv + bb[...], 0.0)
        hcat = jnp.concatenate([ha, hb], axis=1)
        y = jnp.dot(hcat, w_ref[...],
                    preferred_element_type=jnp.float32) * dinv
        oa[...] = y[:, :32]
        ob[...] = y[:, 32:]

    return pl.pallas_call(
        body,
        grid=(n // nb,),
        in_specs=[
            pl.BlockSpec((nb, 32), lambda i: (i, 0)),
            pl.BlockSpec((nb, 32), lambda i: (i, 0)),
            pl.BlockSpec((nb, 32), lambda i: (i, 0)),
            pl.BlockSpec((nb, 32), lambda i: (i, 0)),
            pl.BlockSpec((nb, 1), lambda i: (i, 0)),
            pl.BlockSpec((h, h), lambda i: (0, 0)),
            pl.BlockSpec((1, 32), lambda i: (0, 0)),
            pl.BlockSpec((1, 32), lambda i: (0, 0)),
        ],
        out_specs=[
            pl.BlockSpec((nb, 32), lambda i: (i, 0)),
            pl.BlockSpec((nb, 32), lambda i: (i, 0)),
        ],
        out_shape=[
            jax.ShapeDtypeStruct((n, 32), jnp.float32),
            jax.ShapeDtypeStruct((n, 32), jnp.float32),
        ],
    )(agg_a, agg_b, y_a, y_b, dinv, w, b_a, b_b)


def _tc_final(agg_a, agg_b, y_a, y_b, dinv, b_a, b_b, wl, bl):
    n = agg_a.shape[0]
    h = wl.shape[0]
    dout = wl.shape[1]
    nb = _pick_nb(n)

    def body(aa, ab, ya, yb, dv, ba, bb, w_ref, bl_ref, o):
        dinv = dv[...]
        ha = jnp.maximum((aa[...] + ya[...]) * dinv + ba[...], 0.0)
        hb = jnp.maximum((ab[...] + yb[...]) * dinv + bb[...], 0.0)
        hcat = jnp.concatenate([ha, hb], axis=1)
        o[...] = jnp.dot(hcat, w_ref[...],
                         preferred_element_type=jnp.float32) + bl_ref[...]

    return pl.pallas_call(
        body,
        grid=(n // nb,),
        in_specs=[
            pl.BlockSpec((nb, 32), lambda i: (i, 0)),
            pl.BlockSpec((nb, 32), lambda i: (i, 0)),
            pl.BlockSpec((nb, 32), lambda i: (i, 0)),
            pl.BlockSpec((nb, 32), lambda i: (i, 0)),
            pl.BlockSpec((nb, 1), lambda i: (i, 0)),
            pl.BlockSpec((1, 32), lambda i: (0, 0)),
            pl.BlockSpec((1, 32), lambda i: (0, 0)),
            pl.BlockSpec((h, dout), lambda i: (0, 0)),
            pl.BlockSpec((1, dout), lambda i: (0, 0)),
        ],
        out_specs=pl.BlockSpec((nb, dout), lambda i: (i, 0)),
        out_shape=jax.ShapeDtypeStruct((n, dout), jnp.float32),
    )(agg_a, agg_b, y_a, y_b, dinv, b_a, b_b, wl, bl)


def kernel(x, edge_index, batch, W0, b0, W1, b1, W2, b2, Wl, bl):
    del batch
    n = x.shape[0]
    e = edge_index.shape[1]
    ec = e // CHUNK
    align = NS * LANES
    n_pad = ((n + align - 1) // align) * align

    srcm = edge_index[0].reshape(ec, CHUNK)
    dstm = edge_index[1].reshape(ec, CHUNK)

    degp = _sc_degree(dstm, n_pad)
    degp = degp[:, :n].reshape(2, n, 1)

    y_a, y_b, dinv = _tc_first(x, degp, W0)

    b0a, b0b = b0[:32].reshape(1, 32), b0[32:].reshape(1, 32)
    b1a, b1b = b1[:32].reshape(1, 32), b1[32:].reshape(1, 32)
    b2a, b2b = b2[:32].reshape(1, 32), b2[32:].reshape(1, 32)
    bl2 = bl.reshape(1, -1)

    agg_a, agg_b = _sc_scatter(y_a, y_b, srcm, dstm, n_pad)
    y_a, y_b = _tc_mid(agg_a[:n], agg_b[:n], y_a, y_b, dinv, W1, b0a, b0b)

    agg_a, agg_b = _sc_scatter(y_a, y_b, srcm, dstm, n_pad)
    y_a, y_b = _tc_mid(agg_a[:n], agg_b[:n], y_a, y_b, dinv, W2, b1a, b1b)

    agg_a, agg_b = _sc_scatter(y_a, y_b, srcm, dstm, n_pad)
    return _tc_final(agg_a[:n], agg_b[:n], y_a, y_b, dinv, b2a, b2b, Wl, bl2)


# speedup vs baseline: 18.6048x; 1.8024x over previous
import functools

import jax
import jax.numpy as jnp
from jax import lax
from jax.experimental import pallas as pl
from jax.experimental.pallas import tpu as pltpu
from jax.experimental.pallas import tpu_sc as plsc

NC = 2
NS = 16
LANES = 16
CHUNK = 128
IB = 6


def _mesh():
    return plsc.VectorSubcoreMesh(core_axis_name="c", subcore_axis_name="s",
                                  num_cores=NC, num_subcores=NS)


def _sc_degree(dstm, n_pad):
    ec = dstm.shape[0]
    rpt = ec // (NC * NS)
    nbat = rpt // IB
    slc = n_pad // NS

    @functools.partial(
        pl.kernel,
        out_type=jax.ShapeDtypeStruct((NC * n_pad,), jnp.float32),
        mesh=_mesh(),
        scratch_types=[
            pltpu.VMEM((IB, CHUNK), jnp.int32),
            pltpu.VMEM((CHUNK,), jnp.float32),
            pltpu.VMEM((slc,), jnp.float32),
            pltpu.VMEM_SHARED((n_pad,), jnp.float32),
            pltpu.SemaphoreType.DMA,
        ],
        compiler_params=pltpu.CompilerParams(needs_layout_passes=False,
                                             use_tc_tiling_on_sc=False),
    )
    def k(dstm_hbm, out_hbm, dbuf, obuf, zbuf, sacc, ssem):
        c = lax.axis_index("c")
        t = lax.axis_index("s")

        zeros16 = jnp.zeros((LANES,), jnp.float32)
        ones16 = jnp.ones((LANES,), jnp.float32)
        for g in range(CHUNK // LANES):
            obuf[pl.ds(g * LANES, LANES)] = ones16

        def zbody(i, carry):
            zbuf[pl.ds(i * LANES, LANES)] = zeros16
            return carry
        lax.fori_loop(0, slc // LANES, zbody, 0)
        base = t * slc
        pltpu.sync_copy(zbuf, sacc.at[pl.ds(base, slc)])
        plsc.subcore_barrier()

        row0 = (c * NS + t) * rpt

        def cbody(nb, carry):
            pltpu.sync_copy(dstm_hbm.at[pl.ds(row0 + nb * IB, IB)], dbuf)
            sds = [pltpu.async_copy(obuf, sacc.at[dbuf.at[kk]], ssem, add=True)
                   for kk in range(IB)]
            for d in sds:
                d.wait()
            return carry
        lax.fori_loop(0, nbat, cbody, 0)

        plsc.subcore_barrier()
        pltpu.sync_copy(sacc.at[pl.ds(base, slc)],
                        out_hbm.at[pl.ds(c * n_pad + base, slc)])

    return k(dstm).reshape(NC, n_pad)


def _sc_scatter(y_a, y_b, srcm, dstm, n_pad):
    ec = srcm.shape[0]
    rpt = ec // NS
    nbat = rpt // IB
    slc = n_pad // NS
    nfull = slc // CHUNK
    tail = slc - nfull * CHUNK

    @functools.partial(
        pl.kernel,
        out_type=(jax.ShapeDtypeStruct((n_pad, 32), jnp.float32),
                  jax.ShapeDtypeStruct((n_pad, 32), jnp.float32)),
        mesh=_mesh(),
        scratch_types=[
            pltpu.VMEM((IB, CHUNK), jnp.int32),
            pltpu.VMEM((IB, CHUNK), jnp.int32),
            pltpu.VMEM((IB, CHUNK, 32), jnp.float32),
            pltpu.VMEM((CHUNK, 32), jnp.float32),
            pltpu.VMEM_SHARED((n_pad, 32), jnp.float32),
            pltpu.SemaphoreType.DMA,
            pltpu.SemaphoreType.DMA,
        ],
        compiler_params=pltpu.CompilerParams(needs_layout_passes=False,
                                             use_tc_tiling_on_sc=False),
    )
    def k(ya_hbm, yb_hbm, srcm_hbm, dstm_hbm, out_a, out_b,
          sbuf, dbuf, rows, zbuf, acc, gsem, ssem):
        c = lax.axis_index("c")
        t = lax.axis_index("s")

        zeros16 = jnp.zeros((LANES,), jnp.float32)

        def zbody(i, carry):
            r = i // 2
            zbuf[r, pl.ds((i % 2) * LANES, LANES)] = zeros16
            return carry
        lax.fori_loop(0, CHUNK * 2, zbody, 0)

        base = t * slc
        zds = []
        for q in range(nfull):
            zds.append(pltpu.async_copy(
                zbuf, acc.at[pl.ds(base + q * CHUNK, CHUNK)], ssem))
        if tail:
            zds.append(pltpu.async_copy(
                zbuf.at[pl.ds(0, tail)],
                acc.at[pl.ds(base + nfull * CHUNK, tail)], ssem))
        for d in zds:
            d.wait()
        plsc.subcore_barrier()

        row0 = t * rpt

        def batch_loop(ytab_hbm):
            def body(nb, carry):
                r = row0 + nb * IB
                pltpu.sync_copy(srcm_hbm.at[pl.ds(r, IB)], sbuf)
                pltpu.sync_copy(dstm_hbm.at[pl.ds(r, IB)], dbuf)
                gds = [pltpu.async_copy(ytab_hbm.at[sbuf.at[kk]],
                                        rows.at[kk], gsem)
                       for kk in range(IB)]
                sds = []
                for kk in range(IB):
                    gds[kk].wait()
                    sds.append(pltpu.async_copy(rows.at[kk],
                                                acc.at[dbuf.at[kk]],
                                                ssem, add=True))
                for d in sds:
                    d.wait()
                return carry
            lax.fori_loop(0, nbat, body, 0)

        @pl.when(c == 0)
        def _():
            batch_loop(ya_hbm)

        @pl.when(c == 1)
        def _():
            batch_loop(yb_hbm)

        plsc.subcore_barrier()

        @pl.when(c == 0)
        def _():
            pltpu.sync_copy(acc.at[pl.ds(base, slc)], out_a.at[pl.ds(base, slc)])

        @pl.when(c == 1)
        def _():
            pltpu.sync_copy(acc.at[pl.ds(base, slc)], out_b.at[pl.ds(base, slc)])

    return k(y_a, y_b, srcm, dstm)


def _pick_nb(n):
    for nb in (2000, 2500, 1000, 500, 250, 200, 125, 100, 50, 25, 16, 8):
        if n % nb == 0:
            return nb
    return n


def _tc_first(x, degp, w0):
    n, din = x.shape
    h = w0.shape[1]
    nb = _pick_nb(n)

    def body(x_ref, dp_ref, w_ref, ya_ref, yb_ref, dinv_ref):
        deg = dp_ref[0] + dp_ref[1] + 1.0
        dinv = lax.rsqrt(deg)
        y = jnp.dot(x_ref[...], w_ref[...],
                    preferred_element_type=jnp.float32) * dinv
        ya_ref[...] = y[:, :32]
        yb_ref[...] = y[:, 32:]
        dinv_ref[...] = dinv

    return pl.pallas_call(
        body,
        grid=(n // nb,),
        in_specs=[
            pl.BlockSpec((nb, din), lambda i: (i, 0)),
            pl.BlockSpec((2, nb, 1), lambda i: (0, i, 0)),
            pl.BlockSpec((din, h), lambda i: (0, 0)),
        ],
        out_specs=[
            pl.BlockSpec((nb, 32), lambda i: (i, 0)),
            pl.BlockSpec((nb, 32), lambda i: (i, 0)),
            pl.BlockSpec((nb, 1), lambda i: (i, 0)),
        ],
        out_shape=[
            jax.ShapeDtypeStruct((n, 32), jnp.float32),
            jax.ShapeDtypeStruct((n, 32), jnp.float32),
            jax.ShapeDtypeStruct((n, 1), jnp.float32),
        ],
    )(x, degp, w0)


def _tc_mid(agg_a, agg_b, y_a, y_b, dinv, w, b_a, b_b):
    n = agg_a.shape[0]
    h = w.shape[0]
    nb = _pick_nb(n)

    def body(aa, ab, ya, yb, dv, w_ref, ba, bb, oa, ob):
        dinv = dv[...]
        ha = jnp.maximum((aa[...] + ya[...]) * dinv + ba[...], 0.0)
        hb = jnp.maximum((ab[...] + yb[...]) * dinv + bb[...], 0.0)
        hcat = jnp.concatenate([ha, hb], axis=1)
        y = jnp.dot(hcat, w_ref[...],
                    preferred_element_type=jnp.float32) * dinv
        oa[...] = y[:, :32]
        ob[...] = y[:, 32:]

    return pl.pallas_call(
        body,
        grid=(n // nb,),
        in_specs=[
            pl.BlockSpec((nb, 32), lambda i: (i, 0)),
            pl.BlockSpec((nb, 32), lambda i: (i, 0)),
            pl.BlockSpec((nb, 32), lambda i: (i, 0)),
            pl.BlockSpec((nb, 32), lambda i: (i, 0)),
            pl.BlockSpec((nb, 1), lambda i: (i, 0)),
            pl.BlockSpec((h, h), lambda i: (0, 0)),
            pl.BlockSpec((1, 32), lambda i: (0, 0)),
            pl.BlockSpec((1, 32), lambda i: (0, 0)),
        ],
        out_specs=[
            pl.BlockSpec((nb, 32), lambda i: (i, 0)),
            pl.BlockSpec((nb, 32), lambda i: (i, 0)),
        ],
        out_shape=[
            jax.ShapeDtypeStruct((n, 32), jnp.float32),
            jax.ShapeDtypeStruct((n, 32), jnp.float32),
        ],
    )(agg_a, agg_b, y_a, y_b, dinv, w, b_a, b_b)


def _tc_final(agg_a, agg_b, y_a, y_b, dinv, b_a, b_b, wl, bl):
    n = agg_a.shape[0]
    h = wl.shape[0]
    dout = wl.shape[1]
    nb = _pick_nb(n)

    def body(aa, ab, ya, yb, dv, ba, bb, w_ref, bl_ref, o):
        dinv = dv[...]
        ha = jnp.maximum((aa[...] + ya[...]) * dinv + ba[...], 0.0)
        hb = jnp.maximum((ab[...] + yb[...]) * dinv + bb[...], 0.0)
        hcat = jnp.concatenate([ha, hb], axis=1)
        o[...] = jnp.dot(hcat, w_ref[...],
                         preferred_element_type=jnp.float32) + bl_ref[...]

    return pl.pallas_call(
        body,
        grid=(n // nb,),
        in_specs=[
            pl.BlockSpec((nb, 32), lambda i: (i, 0)),
            pl.BlockSpec((nb, 32), lambda i: (i, 0)),
            pl.BlockSpec((nb, 32), lambda i: (i, 0)),
            pl.BlockSpec((nb, 32), lambda i: (i, 0)),
            pl.BlockSpec((nb, 1), lambda i: (i, 0)),
            pl.BlockSpec((1, 32), lambda i: (0, 0)),
            pl.BlockSpec((1, 32), lambda i: (0, 0)),
            pl.BlockSpec((h, dout), lambda i: (0, 0)),
            pl.BlockSpec((1, dout), lambda i: (0, 0)),
        ],
        out_specs=pl.BlockSpec((nb, dout), lambda i: (i, 0)),
        out_shape=jax.ShapeDtypeStruct((n, dout), jnp.float32),
    )(agg_a, agg_b, y_a, y_b, dinv, b_a, b_b, wl, bl)


def kernel(x, edge_index, batch, W0, b0, W1, b1, W2, b2, Wl, bl):
    del batch
    n = x.shape[0]
    e = edge_index.shape[1]
    align = NS * LANES
    n_pad = ((n + align - 1) // align) * align
    if n_pad == n:
        n_pad += align

    grain = CHUNK * NC * NS * IB
    e_pad = ((e + grain - 1) // grain) * grain
    src = edge_index[0]
    dst = edge_index[1]
    if e_pad != e:
        src = jnp.concatenate([src, jnp.zeros((e_pad - e,), jnp.int32)])
        dst = jnp.concatenate([dst, jnp.full((e_pad - e,), n, jnp.int32)])
    srcm = src.reshape(e_pad // CHUNK, CHUNK)
    dstm = dst.reshape(e_pad // CHUNK, CHUNK)

    degp = _sc_degree(dstm, n_pad)
    degp = degp[:, :n].reshape(2, n, 1)

    y_a, y_b, dinv = _tc_first(x, degp, W0)

    b0a, b0b = b0[:32].reshape(1, 32), b0[32:].reshape(1, 32)
    b1a, b1b = b1[:32].reshape(1, 32), b1[32:].reshape(1, 32)
    b2a, b2b = b2[:32].reshape(1, 32), b2[32:].reshape(1, 32)
    bl2 = bl.reshape(1, -1)

    agg_a, agg_b = _sc_scatter(y_a, y_b, srcm, dstm, n_pad)
    y_a, y_b = _tc_mid(agg_a[:n], agg_b[:n], y_a, y_b, dinv, W1, b0a, b0b)

    agg_a, agg_b = _sc_scatter(y_a, y_b, srcm, dstm, n_pad)
    y_a, y_b = _tc_mid(agg_a[:n], agg_b[:n], y_a, y_b, dinv, W2, b1a, b1b)

    agg_a, agg_b = _sc_scatter(y_a, y_b, srcm, dstm, n_pad)
    return _tc_final(agg_a[:n], agg_b[:n], y_a, y_b, dinv, b2a, b2b, Wl, bl2)
